# baseline (speedup 1.0000x reference)
import jax
import jax.numpy as jnp
from jax.experimental import pallas as pl
from jax.experimental.pallas import tpu as pltpu

_BM = 18000
_BO = 128


def _linear_kernel(x_ref, w_ref, o_ref):
    o_ref[...] = jax.lax.dot_general(
        x_ref[...],
        w_ref[...],
        dimension_numbers=(((1,), (1,)), ((), ())),
        preferred_element_type=jnp.float32,
    )


def kernel(x, W):
    M, K = x.shape
    O = W.shape[0]
    return pl.pallas_call(
        _linear_kernel,
        grid=(pl.cdiv(M, _BM), O // _BO),
        in_specs=[
            pl.BlockSpec((_BM, K), lambda i, j: (i, 0)),
            pl.BlockSpec((_BO, K), lambda i, j: (j, 0)),
        ],
        out_specs=pl.BlockSpec((_BM, _BO), lambda i, j: (i, j)),
        out_shape=jax.ShapeDtypeStruct((M, O), jnp.float32),
        compiler_params=pltpu.CompilerParams(
            dimension_semantics=("parallel", "parallel"),
        ),
    )(x, W)


# speedup vs baseline: 1.4247x; 1.4247x over previous
import jax
import jax.numpy as jnp
from jax.experimental import pallas as pl
from jax.experimental.pallas import tpu as pltpu

_BM = 14912


def _linear_kernel(x_ref, w_ref, o_ref):
    o_ref[...] = jax.lax.dot_general(
        x_ref[...],
        w_ref[...],
        dimension_numbers=(((1,), (1,)), ((), ())),
        preferred_element_type=jnp.float32,
    )


def kernel(x, W):
    M, K = x.shape
    O = W.shape[0]
    return pl.pallas_call(
        _linear_kernel,
        grid=(pl.cdiv(M, _BM),),
        in_specs=[
            pl.BlockSpec((_BM, K), lambda i: (i, 0)),
            pl.BlockSpec((O, K), lambda i: (0, 0)),
        ],
        out_specs=pl.BlockSpec((_BM, O), lambda i: (i, 0)),
        out_shape=jax.ShapeDtypeStruct((M, O), jnp.float32),
        compiler_params=pltpu.CompilerParams(
            dimension_semantics=("parallel",),
        ),
    )(x, W)
